# Initial kernel scaffold; baseline (speedup 1.0000x reference)
import jax
import jax.numpy as jnp
from jax.experimental import pallas as pl


def kernel(user_table, item_table, edge_weight, edge_src, edge_dst, users, items):
    raise NotImplementedError("write your pallas kernel here")


# speedup vs baseline: 10.6873x; 10.6873x over previous
import functools

import jax
import jax.numpy as jnp
from jax import lax
from jax.experimental import pallas as pl
from jax.experimental.pallas import tpu as pltpu
from jax.experimental.pallas import tpu_sc as plsc

N_USERS = 50000
N_ITEMS = 50000
N_NODES = N_USERS + N_ITEMS
N_EDGES = 3200000
D = 16
NS = 16
C = 1024
SUB = 128
NSUB = C // SUB
N_CHUNKS = N_EDGES // C
FULL_ZBLK = N_NODES // C
TAIL_ROWS = N_NODES - FULL_ZBLK * C


def _prop_body(em0, esrc, edst2, ew, users, items, o1, o2, o3, selo,
               acc, src_v, dst_v, w_v, rows, idx_sel,
               gsem, ssem):
    wid = lax.axis_index("s")

    nch = jnp.where(wid < N_CHUNKS - (N_CHUNKS // NS) * NS,
                    N_CHUNKS // NS + 1, N_CHUNKS // NS).astype(jnp.int32)

    tables = (em0, o1, o2)
    outs = (o1, o2, o3)
    for l in range(3):
        tbl = tables[l]
        out_t = outs[l]

        @pl.loop(0, C)
        def _zb(j):
            rows[j] = jnp.zeros((D,), jnp.float32)

        for k in range(FULL_ZBLK // NS + 1):
            b = wid + NS * k

            @pl.when(b < FULL_ZBLK)
            def _():
                pltpu.sync_copy(rows, acc.at[pl.ds(b * C, C)])

            @pl.when(b == FULL_ZBLK)
            def _():
                pltpu.sync_copy(rows.at[pl.ds(0, TAIL_ROWS)],
                                acc.at[pl.ds(FULL_ZBLK * C, TAIL_ROWS)])
        plsc.subcore_barrier()

        @pl.loop(0, nch)
        def _chunk(k):
            c = wid + NS * k
            base = c * C
            pltpu.sync_copy(esrc.at[pl.ds(base, C)], src_v)
            pltpu.sync_copy(edst2.at[pl.ds(c * NSUB, NSUB)], dst_v)
            pltpu.sync_copy(ew.at[pl.ds(base, C)], w_v)
            gds = [
                pltpu.async_copy(tbl.at[src_v.at[pl.ds(s * SUB, SUB)]],
                                 rows.at[pl.ds(s * SUB, SUB)], gsem)
                for s in range(NSUB)
            ]
            for dsc in gds:
                dsc.wait()

            @pl.loop(0, C // D)
            def _grp(g):
                for i in range(D):
                    j = g * D + i
                    bw = plsc.load_gather(w_v, [jnp.full((D,), j, jnp.int32)])
                    rows[j] = rows[j] * bw

            sds = [
                pltpu.async_copy(rows.at[pl.ds(s * SUB, SUB)],
                                 acc.at[dst_v.at[s]], ssem, add=True)
                for s in range(NSUB)
            ]
            for dsc in sds:
                dsc.wait()

        plsc.subcore_barrier()

        for k in range(FULL_ZBLK // NS + 1):
            b = wid + NS * k

            @pl.when(b < FULL_ZBLK)
            def _():
                pltpu.sync_copy(acc.at[pl.ds(b * C, C)],
                                out_t.at[pl.ds(b * C, C)])

            @pl.when(b == FULL_ZBLK)
            def _():
                pltpu.sync_copy(acc.at[pl.ds(FULL_ZBLK * C, TAIL_ROWS)],
                                out_t.at[pl.ds(FULL_ZBLK * C, TAIL_ROWS)])
        plsc.subcore_barrier()

    @pl.when(wid < 8)
    def _():
        pltpu.sync_copy(users.at[pl.ds(wid * SUB, SUB)], idx_sel)

    @pl.when(wid >= 8)
    def _():
        pltpu.sync_copy(items.at[pl.ds((wid - 8) * SUB, SUB)], idx_sel)

    off = jnp.where(wid < 8, 0, N_USERS).astype(jnp.int32)
    for g in range(SUB // D):
        idx_sel[pl.ds(g * D, D)] = idx_sel[pl.ds(g * D, D)] + off

    gds = [
        pltpu.async_copy(t.at[idx_sel], rows.at[pl.ds(e * SUB, SUB)], gsem)
        for e, t in enumerate((em0, o1, o2, o3))
    ]
    for dsc in gds:
        dsc.wait()

    @pl.loop(0, SUB)
    def _mean(r):
        v = (rows[r] + rows[SUB + r] + rows[2 * SUB + r]
             + rows[3 * SUB + r]) * 0.25
        rows[r] = v

    pltpu.sync_copy(rows.at[pl.ds(0, SUB)], selo.at[pl.ds(wid * SUB, SUB)])


_prop = functools.partial(
    pl.kernel,
    out_type=[
        jax.ShapeDtypeStruct((N_NODES, D), jnp.float32),
        jax.ShapeDtypeStruct((N_NODES, D), jnp.float32),
        jax.ShapeDtypeStruct((N_NODES, D), jnp.float32),
        jax.ShapeDtypeStruct((2048, D), jnp.float32),
    ],
    mesh=plsc.VectorSubcoreMesh(core_axis_name="c", subcore_axis_name="s",
                                num_cores=1),
    compiler_params=pltpu.CompilerParams(needs_layout_passes=False,
                                         use_tc_tiling_on_sc=False),
    scratch_types=[
        pltpu.VMEM_SHARED((N_NODES, D), jnp.float32),
        pltpu.VMEM((C,), jnp.int32),
        pltpu.VMEM((NSUB, SUB), jnp.int32),
        pltpu.VMEM((C,), jnp.float32),
        pltpu.VMEM((C, D), jnp.float32),
        pltpu.VMEM((SUB,), jnp.int32),
        pltpu.SemaphoreType.DMA,
        pltpu.SemaphoreType.DMA,
    ],
)(_prop_body)


def _ratings_body(u_ref, it_ref, out_ref):
    out_ref[...] = jax.nn.sigmoid(
        jnp.dot(u_ref[...], it_ref[...], preferred_element_type=jnp.float32))


_ratings = pl.pallas_call(
    _ratings_body,
    out_shape=jax.ShapeDtypeStruct((1024, 1024), jnp.float32),
)


def kernel(user_table, item_table, edge_weight, edge_src, edge_dst, users, items):
    em0 = jnp.concatenate([user_table, item_table], axis=0)
    edst2 = edge_dst.reshape(N_EDGES // SUB, SUB)
    _o1, _o2, _o3, sel = _prop(em0, edge_src, edst2, edge_weight, users, items)
    u = sel[:1024]
    it_t = sel[1024:].T
    return _ratings(u, it_t)


# speedup vs baseline: 15.3724x; 1.4384x over previous
import functools

import jax
import jax.numpy as jnp
from jax import lax
from jax.experimental import pallas as pl
from jax.experimental.pallas import tpu as pltpu
from jax.experimental.pallas import tpu_sc as plsc

N_USERS = 50000
N_ITEMS = 50000
N_NODES = N_USERS + N_ITEMS
N_EDGES = 3200000
D = 16
NS = 16
C = 512
SUB = 128
NSUB = C // SUB
NCH_T = 392
N_EDGES_PAD = NS * NCH_T * C
FULL_ZBLK = N_NODES // C
TAIL_ROWS = N_NODES - FULL_ZBLK * C


def _prop_body(em0, esrc, edst2, ew, users, items, o1, o2, o3, selo,
               acc, src0, src1, w0, w1, dst0, dst1, rows0, rows1, idx_sel,
               esem, dsem, gsem, ssem):
    wid = lax.axis_index("s")
    tile0 = wid * NCH_T

    def gather_issue(tbl, src_v, rows_v):
        for s in range(NSUB):
            pltpu.async_copy(tbl.at[src_v.at[pl.ds(s * SUB, SUB)]],
                             rows_v.at[pl.ds(s * SUB, SUB)], gsem)

    def gather_wait(tbl, src_v, rows_v):
        for s in range(NSUB):
            pltpu.make_async_copy(tbl.at[src_v.at[pl.ds(s * SUB, SUB)]],
                                  rows_v.at[pl.ds(s * SUB, SUB)], gsem).wait()

    def scatter_issue(rows_v, dst_v):
        for s in range(NSUB):
            pltpu.async_copy(rows_v.at[pl.ds(s * SUB, SUB)],
                             acc.at[dst_v.at[s]], ssem, add=True)

    def scatter_wait(rows_v, dst_v):
        for s in range(NSUB):
            pltpu.make_async_copy(rows_v.at[pl.ds(s * SUB, SUB)],
                                  acc.at[dst_v.at[s]], ssem).wait()

    def srcw_issue(c, src_v, w_v):
        pltpu.async_copy(esrc.at[pl.ds(c * C, C)], src_v, esem)
        pltpu.async_copy(ew.at[pl.ds(c * C, C)], w_v, esem)

    def srcw_wait(c, src_v, w_v):
        pltpu.make_async_copy(esrc.at[pl.ds(c * C, C)], src_v, esem).wait()
        pltpu.make_async_copy(ew.at[pl.ds(c * C, C)], w_v, esem).wait()

    def compute(rows_v, w_v):
        @pl.loop(0, C // D)
        def _g(g):
            for i in range(D):
                j = g * D + i
                bw = plsc.load_gather(w_v, [jnp.full((D,), j, jnp.int32)])
                rows_v[j] = rows_v[j] * bw

    tables = (em0, o1, o2)
    outs = (o1, o2, o3)
    for l in range(3):
        tbl = tables[l]
        out_t = outs[l]

        @pl.loop(0, C)
        def _zb(j):
            rows0[j] = jnp.zeros((D,), jnp.float32)

        for k in range(FULL_ZBLK // NS + 1):
            b = wid + NS * k

            @pl.when(b < FULL_ZBLK)
            def _():
                pltpu.sync_copy(rows0, acc.at[pl.ds(b * C, C)])

            @pl.when(b == FULL_ZBLK)
            def _():
                pltpu.sync_copy(rows0.at[pl.ds(0, TAIL_ROWS)],
                                acc.at[pl.ds(FULL_ZBLK * C, TAIL_ROWS)])
        plsc.subcore_barrier()

        pltpu.sync_copy(esrc.at[pl.ds(tile0 * C, C)], src0)
        pltpu.sync_copy(ew.at[pl.ds(tile0 * C, C)], w0)
        pltpu.sync_copy(edst2.at[pl.ds(tile0 * NSUB, NSUB)], dst0)
        gather_issue(tbl, src0, rows0)
        srcw_issue(tile0 + 1, src1, w1)

        @pl.loop(0, NCH_T, step=2)
        def _pair(k2):
            c = tile0 + k2
            gather_wait(tbl, src0, rows0)

            @pl.when(k2 > 0)
            def _():
                scatter_wait(rows1, dst1)

            pltpu.async_copy(edst2.at[pl.ds((c + 1) * NSUB, NSUB)], dst1, dsem)
            srcw_wait(c + 1, src1, w1)
            gather_issue(tbl, src1, rows1)

            @pl.when(k2 > 0)
            def _():
                pltpu.make_async_copy(edst2.at[pl.ds(c * NSUB, NSUB)],
                                      dst0, dsem).wait()

            compute(rows0, w0)
            scatter_issue(rows0, dst0)

            @pl.when(k2 < NCH_T - 2)
            def _():
                srcw_issue(c + 2, src0, w0)

            gather_wait(tbl, src1, rows1)
            scatter_wait(rows0, dst0)

            @pl.when(k2 < NCH_T - 2)
            def _():
                pltpu.async_copy(edst2.at[pl.ds((c + 2) * NSUB, NSUB)],
                                 dst0, dsem)
                srcw_wait(c + 2, src0, w0)
                gather_issue(tbl, src0, rows0)

            pltpu.make_async_copy(edst2.at[pl.ds((c + 1) * NSUB, NSUB)],
                                  dst1, dsem).wait()
            compute(rows1, w1)
            scatter_issue(rows1, dst1)

            @pl.when(k2 < NCH_T - 2)
            def _():
                srcw_issue(c + 3, src1, w1)

        scatter_wait(rows1, dst1)
        plsc.subcore_barrier()

        for k in range(FULL_ZBLK // NS + 1):
            b = wid + NS * k

            @pl.when(b < FULL_ZBLK)
            def _():
                pltpu.sync_copy(acc.at[pl.ds(b * C, C)],
                                out_t.at[pl.ds(b * C, C)])

            @pl.when(b == FULL_ZBLK)
            def _():
                pltpu.sync_copy(acc.at[pl.ds(FULL_ZBLK * C, TAIL_ROWS)],
                                out_t.at[pl.ds(FULL_ZBLK * C, TAIL_ROWS)])
        plsc.subcore_barrier()

    @pl.when(wid < 8)
    def _():
        pltpu.sync_copy(users.at[pl.ds(wid * SUB, SUB)], idx_sel)

    @pl.when(wid >= 8)
    def _():
        pltpu.sync_copy(items.at[pl.ds((wid - 8) * SUB, SUB)], idx_sel)

    off = jnp.where(wid < 8, 0, N_USERS).astype(jnp.int32)
    for g in range(SUB // D):
        idx_sel[pl.ds(g * D, D)] = idx_sel[pl.ds(g * D, D)] + off

    gds = [
        pltpu.async_copy(t.at[idx_sel], rows0.at[pl.ds(e * SUB, SUB)], gsem)
        for e, t in enumerate((em0, o1, o2, o3))
    ]
    for dsc in gds:
        dsc.wait()

    @pl.loop(0, SUB)
    def _mean(r):
        v = (rows0[r] + rows0[SUB + r] + rows0[2 * SUB + r]
             + rows0[3 * SUB + r]) * 0.25
        rows0[r] = v

    pltpu.sync_copy(rows0.at[pl.ds(0, SUB)], selo.at[pl.ds(wid * SUB, SUB)])


_prop = functools.partial(
    pl.kernel,
    out_type=[
        jax.ShapeDtypeStruct((N_NODES, D), jnp.float32),
        jax.ShapeDtypeStruct((N_NODES, D), jnp.float32),
        jax.ShapeDtypeStruct((N_NODES, D), jnp.float32),
        jax.ShapeDtypeStruct((2048, D), jnp.float32),
    ],
    mesh=plsc.VectorSubcoreMesh(core_axis_name="c", subcore_axis_name="s",
                                num_cores=1),
    compiler_params=pltpu.CompilerParams(needs_layout_passes=False,
                                         use_tc_tiling_on_sc=False),
    scratch_types=[
        pltpu.VMEM_SHARED((N_NODES, D), jnp.float32),
        pltpu.VMEM((C,), jnp.int32),
        pltpu.VMEM((C,), jnp.int32),
        pltpu.VMEM((C,), jnp.float32),
        pltpu.VMEM((C,), jnp.float32),
        pltpu.VMEM((NSUB, SUB), jnp.int32),
        pltpu.VMEM((NSUB, SUB), jnp.int32),
        pltpu.VMEM((C, D), jnp.float32),
        pltpu.VMEM((C, D), jnp.float32),
        pltpu.VMEM((SUB,), jnp.int32),
        pltpu.SemaphoreType.DMA,
        pltpu.SemaphoreType.DMA,
        pltpu.SemaphoreType.DMA,
        pltpu.SemaphoreType.DMA,
    ],
)(_prop_body)


def _ratings_body(u_ref, it_ref, out_ref):
    out_ref[...] = jax.nn.sigmoid(
        jnp.dot(u_ref[...], it_ref[...], preferred_element_type=jnp.float32))


_ratings = pl.pallas_call(
    _ratings_body,
    out_shape=jax.ShapeDtypeStruct((1024, 1024), jnp.float32),
)


def kernel(user_table, item_table, edge_weight, edge_src, edge_dst, users, items):
    em0 = jnp.concatenate([user_table, item_table], axis=0)
    pad = N_EDGES_PAD - N_EDGES
    esrc_p = jnp.concatenate([edge_src, jnp.zeros((pad,), jnp.int32)])
    ew_p = jnp.concatenate([edge_weight, jnp.zeros((pad,), jnp.float32)])
    edst_p = jnp.concatenate([edge_dst, jnp.zeros((pad,), jnp.int32)])
    edst2 = edst_p.reshape(N_EDGES_PAD // SUB, SUB)
    _o1, _o2, _o3, sel = _prop(em0, esrc_p, edst2, ew_p, users, items)
    u = sel[:1024]
    it_t = sel[1024:].T
    return _ratings(u, it_t)


# speedup vs baseline: 34.8749x; 2.2687x over previous
import functools

import jax
import jax.numpy as jnp
from jax import lax
from jax.experimental import pallas as pl
from jax.experimental.pallas import tpu as pltpu
from jax.experimental.pallas import tpu_sc as plsc

N_USERS = 50000
N_ITEMS = 50000
N_NODES = N_USERS + N_ITEMS
N_EDGES = 3200000
D = 16
NS = 16
C = 512
SUB = 128
NSUB = C // SUB
NCH_T = 392
N_EDGES_PAD = NS * NCH_T * C
FULL_ZBLK = N_NODES // C
TAIL_ROWS = N_NODES - FULL_ZBLK * C


def _prop_body(em0, esrc, edst2, ew, users, items, o1, o2, o3, selo,
               acc, src0, src1, w0, w1, dst0, dst1, rows0, rows1, idx_sel,
               esem, dsem, gsem, ssem):
    wid = lax.axis_index("s")
    tile0 = wid * NCH_T

    def gather_issue(tbl, src_v, rows_v):
        for s in range(NSUB):
            pltpu.async_copy(tbl.at[src_v.at[pl.ds(s * SUB, SUB)]],
                             rows_v.at[pl.ds(s * SUB, SUB)], gsem)

    def gather_wait(tbl, src_v, rows_v):
        for s in range(NSUB):
            pltpu.make_async_copy(tbl.at[src_v.at[pl.ds(s * SUB, SUB)]],
                                  rows_v.at[pl.ds(s * SUB, SUB)], gsem).wait()

    def scatter_issue(rows_v, dst_v):
        for s in range(NSUB):
            pltpu.async_copy(rows_v.at[pl.ds(s * SUB, SUB)],
                             acc.at[dst_v.at[s]], ssem, add=True)

    def scatter_wait(rows_v, dst_v):
        for s in range(NSUB):
            pltpu.make_async_copy(rows_v.at[pl.ds(s * SUB, SUB)],
                                  acc.at[dst_v.at[s]], ssem).wait()

    def srcw_issue(c, src_v, w_v):
        pltpu.async_copy(esrc.at[pl.ds(c * C, C)], src_v, esem)
        pltpu.async_copy(ew.at[pl.ds(c * C, C)], w_v, esem)

    def srcw_wait(c, src_v, w_v):
        pltpu.make_async_copy(esrc.at[pl.ds(c * C, C)], src_v, esem).wait()
        pltpu.make_async_copy(ew.at[pl.ds(c * C, C)], w_v, esem).wait()

    def compute(rows_v, w_v):
        @pl.loop(0, C // D)
        def _g(g):
            w16 = w_v[pl.ds(g * D, D)]
            for i in range(D):
                bw = jnp.take_along_axis(
                    w16, jnp.full((D,), i, jnp.int32), axis=0,
                    mode="promise_in_bounds")
                j = g * D + i
                rows_v[j] = rows_v[j] * bw

    tables = (em0, o1, o2)
    outs = (o1, o2, o3)
    for l in range(3):
        tbl = tables[l]
        out_t = outs[l]

        @pl.loop(0, C)
        def _zb(j):
            rows0[j] = jnp.zeros((D,), jnp.float32)

        for k in range(FULL_ZBLK // NS + 1):
            b = wid + NS * k

            @pl.when(b < FULL_ZBLK)
            def _():
                pltpu.sync_copy(rows0, acc.at[pl.ds(b * C, C)])

            @pl.when(b == FULL_ZBLK)
            def _():
                pltpu.sync_copy(rows0.at[pl.ds(0, TAIL_ROWS)],
                                acc.at[pl.ds(FULL_ZBLK * C, TAIL_ROWS)])
        plsc.subcore_barrier()

        pltpu.sync_copy(esrc.at[pl.ds(tile0 * C, C)], src0)
        pltpu.sync_copy(ew.at[pl.ds(tile0 * C, C)], w0)
        pltpu.sync_copy(edst2.at[pl.ds(tile0 * NSUB, NSUB)], dst0)
        gather_issue(tbl, src0, rows0)
        srcw_issue(tile0 + 1, src1, w1)

        @pl.loop(0, NCH_T, step=2)
        def _pair(k2):
            c = tile0 + k2
            gather_wait(tbl, src0, rows0)

            @pl.when(k2 > 0)
            def _():
                scatter_wait(rows1, dst1)

            pltpu.async_copy(edst2.at[pl.ds((c + 1) * NSUB, NSUB)], dst1, dsem)
            srcw_wait(c + 1, src1, w1)
            gather_issue(tbl, src1, rows1)

            @pl.when(k2 > 0)
            def _():
                pltpu.make_async_copy(edst2.at[pl.ds(c * NSUB, NSUB)],
                                      dst0, dsem).wait()

            compute(rows0, w0)
            scatter_issue(rows0, dst0)

            @pl.when(k2 < NCH_T - 2)
            def _():
                srcw_issue(c + 2, src0, w0)

            gather_wait(tbl, src1, rows1)
            scatter_wait(rows0, dst0)

            @pl.when(k2 < NCH_T - 2)
            def _():
                pltpu.async_copy(edst2.at[pl.ds((c + 2) * NSUB, NSUB)],
                                 dst0, dsem)
                srcw_wait(c + 2, src0, w0)
                gather_issue(tbl, src0, rows0)

            pltpu.make_async_copy(edst2.at[pl.ds((c + 1) * NSUB, NSUB)],
                                  dst1, dsem).wait()
            compute(rows1, w1)
            scatter_issue(rows1, dst1)

            @pl.when(k2 < NCH_T - 2)
            def _():
                srcw_issue(c + 3, src1, w1)

        scatter_wait(rows1, dst1)
        plsc.subcore_barrier()

        for k in range(FULL_ZBLK // NS + 1):
            b = wid + NS * k

            @pl.when(b < FULL_ZBLK)
            def _():
                pltpu.sync_copy(acc.at[pl.ds(b * C, C)],
                                out_t.at[pl.ds(b * C, C)])

            @pl.when(b == FULL_ZBLK)
            def _():
                pltpu.sync_copy(acc.at[pl.ds(FULL_ZBLK * C, TAIL_ROWS)],
                                out_t.at[pl.ds(FULL_ZBLK * C, TAIL_ROWS)])
        plsc.subcore_barrier()

    @pl.when(wid < 8)
    def _():
        pltpu.sync_copy(users.at[pl.ds(wid * SUB, SUB)], idx_sel)

    @pl.when(wid >= 8)
    def _():
        pltpu.sync_copy(items.at[pl.ds((wid - 8) * SUB, SUB)], idx_sel)

    off = jnp.where(wid < 8, 0, N_USERS).astype(jnp.int32)
    for g in range(SUB // D):
        idx_sel[pl.ds(g * D, D)] = idx_sel[pl.ds(g * D, D)] + off

    gds = [
        pltpu.async_copy(t.at[idx_sel], rows0.at[pl.ds(e * SUB, SUB)], gsem)
        for e, t in enumerate((em0, o1, o2, o3))
    ]
    for dsc in gds:
        dsc.wait()

    @pl.loop(0, SUB)
    def _mean(r):
        v = (rows0[r] + rows0[SUB + r] + rows0[2 * SUB + r]
             + rows0[3 * SUB + r]) * 0.25
        rows0[r] = v

    pltpu.sync_copy(rows0.at[pl.ds(0, SUB)], selo.at[pl.ds(wid * SUB, SUB)])


_prop = functools.partial(
    pl.kernel,
    out_type=[
        jax.ShapeDtypeStruct((N_NODES, D), jnp.float32),
        jax.ShapeDtypeStruct((N_NODES, D), jnp.float32),
        jax.ShapeDtypeStruct((N_NODES, D), jnp.float32),
        jax.ShapeDtypeStruct((2048, D), jnp.float32),
    ],
    mesh=plsc.VectorSubcoreMesh(core_axis_name="c", subcore_axis_name="s",
                                num_cores=1),
    compiler_params=pltpu.CompilerParams(needs_layout_passes=False,
                                         use_tc_tiling_on_sc=False),
    scratch_types=[
        pltpu.VMEM_SHARED((N_NODES, D), jnp.float32),
        pltpu.VMEM((C,), jnp.int32),
        pltpu.VMEM((C,), jnp.int32),
        pltpu.VMEM((C,), jnp.float32),
        pltpu.VMEM((C,), jnp.float32),
        pltpu.VMEM((NSUB, SUB), jnp.int32),
        pltpu.VMEM((NSUB, SUB), jnp.int32),
        pltpu.VMEM((C, D), jnp.float32),
        pltpu.VMEM((C, D), jnp.float32),
        pltpu.VMEM((SUB,), jnp.int32),
        pltpu.SemaphoreType.DMA,
        pltpu.SemaphoreType.DMA,
        pltpu.SemaphoreType.DMA,
        pltpu.SemaphoreType.DMA,
    ],
)(_prop_body)


def _ratings_body(u_ref, it_ref, out_ref):
    out_ref[...] = jax.nn.sigmoid(
        jnp.dot(u_ref[...], it_ref[...], preferred_element_type=jnp.float32))


_ratings = pl.pallas_call(
    _ratings_body,
    out_shape=jax.ShapeDtypeStruct((1024, 1024), jnp.float32),
)


def kernel(user_table, item_table, edge_weight, edge_src, edge_dst, users, items):
    em0 = jnp.concatenate([user_table, item_table], axis=0)
    pad = N_EDGES_PAD - N_EDGES
    esrc_p = jnp.concatenate([edge_src, jnp.zeros((pad,), jnp.int32)])
    ew_p = jnp.concatenate([edge_weight, jnp.zeros((pad,), jnp.float32)])
    edst_p = jnp.concatenate([edge_dst, jnp.zeros((pad,), jnp.int32)])
    edst2 = edst_p.reshape(N_EDGES_PAD // SUB, SUB)
    _o1, _o2, _o3, sel = _prop(em0, esrc_p, edst2, ew_p, users, items)
    u = sel[:1024]
    it_t = sel[1024:].T
    return _ratings(u, it_t)
